# Initial kernel scaffold; baseline (speedup 1.0000x reference)
import jax
import jax.numpy as jnp
from jax.experimental import pallas as pl


def kernel(x, edge_index, edge_weight, W1, b1, W2, b2):
    raise NotImplementedError("write your pallas kernel here")


# speedup vs baseline: 14.1412x; 14.1412x over previous
import functools

import jax
import jax.numpy as jnp
from jax import lax
from jax.experimental import pallas as pl
from jax.experimental.pallas import tpu as pltpu
from jax.experimental.pallas import tpu_sc as plsc

N = 10000
E = 320000
NPAD = 10240
NC = 2
NS = 16
NRP = NPAD // NS
EP = E // NS

_mesh = plsc.VectorSubcoreMesh(
    core_axis_name="c", subcore_axis_name="s", num_cores=NC, num_subcores=NS
)
_sc_params = pltpu.CompilerParams(
    needs_layout_passes=False, use_tc_tiling_on_sc=False
)


CHD = 2000
EPD = E // (NC * NS)


@functools.partial(
    pl.kernel,
    out_type=jax.ShapeDtypeStruct((NC, NPAD), jnp.float32),
    mesh=_mesh,
    compiler_params=_sc_params,
    scratch_types=[
        pltpu.VMEM((CHD,), jnp.int32),
        pltpu.VMEM((CHD,), jnp.float32),
        pltpu.VMEM_SHARED((NPAD,), jnp.float32),
    ],
)
def _deg_kernel(col_h, ew_h, z_h, deg_h, col_v, ew_v, deg_sh):
    c = lax.axis_index("c")
    s = lax.axis_index("s")
    nd = NPAD // NS
    pltpu.sync_copy(z_h.at[pl.ds(s * nd, nd)], deg_sh.at[pl.ds(s * nd, nd)])
    plsc.subcore_barrier()
    base = (c * NS + s) * EPD
    for k in range(EPD // CHD):
        off = pl.multiple_of(base + k * CHD, 8)
        pltpu.sync_copy(col_h.at[pl.ds(off, CHD)], col_v)
        pltpu.sync_copy(ew_h.at[pl.ds(off, CHD)], ew_v)
        pltpu.sync_copy(ew_v, deg_sh.at[col_v], add=True)
    plsc.subcore_barrier()
    pltpu.sync_copy(deg_sh.at[pl.ds(s * nd, nd)], deg_h.at[c, pl.ds(s * nd, nd)])


def _make_agg(D, CH):
    nchunks = EP // CH

    @functools.partial(
        pl.kernel,
        out_type=jax.ShapeDtypeStruct((NC, NPAD, D), jnp.float32),
        mesh=_mesh,
        compiler_params=_sc_params,
        scratch_types=[
            pltpu.VMEM((CH,), jnp.int32),
            pltpu.VMEM((CH,), jnp.int32),
            pltpu.VMEM((CH,), jnp.float32),
            pltpu.VMEM((CH, D), jnp.float32),
            pltpu.VMEM_SHARED((NPAD, D), jnp.float32),
            pltpu.SemaphoreType.DMA,
        ],
    )
    def agg(row_h, col_h, ew_h, pa_h, pb_h, z_h, acc_h, row_v, col_v, ew_v,
            rows_v, acc_sh, sem):
        c = lax.axis_index("c")
        s = lax.axis_index("s")
        pltpu.sync_copy(z_h.at[pl.ds(s * NRP, NRP)],
                        acc_sh.at[pl.ds(s * NRP, NRP)])
        plsc.subcore_barrier()
        base = s * EP
        for k in range(nchunks):
            off = pl.multiple_of(base + k * CH, 8)
            pltpu.sync_copy(row_h.at[pl.ds(off, CH)], row_v)
            pltpu.sync_copy(col_h.at[pl.ds(off, CH)], col_v)
            pltpu.sync_copy(ew_h.at[pl.ds(off, CH)], ew_v)

            @pl.when(c == 0)
            def _():
                pltpu.async_copy(pa_h.at[row_v], rows_v, sem).wait()

            @pl.when(c == 1)
            def _():
                pltpu.async_copy(pb_h.at[row_v], rows_v, sem).wait()

            def mul_body(e, carry):
                wv = plsc.load_gather(ew_v, [jnp.full((16,), e, jnp.int32)])
                for fb in range(D // 16):
                    rows_v[e, pl.ds(fb * 16, 16)] = (
                        rows_v[e, pl.ds(fb * 16, 16)] * wv
                    )
                return carry

            lax.fori_loop(0, CH, mul_body, 0)
            pltpu.sync_copy(rows_v, acc_sh.at[col_v], add=True)
        plsc.subcore_barrier()
        pltpu.sync_copy(acc_sh.at[pl.ds(s * NRP, NRP)],
                        acc_h.at[c, pl.ds(s * NRP, NRP)])

    return agg


_agg64 = _make_agg(64, 1000)
_agg32 = _make_agg(32, 2000)


BN = 512
GRID = NPAD // BN


def _tc1_body(deg0_ref, deg1_ref, x_ref, w1_ref, p_ref, dinv_ref):
    deg = deg0_ref[...] + deg1_ref[...] + 1.0
    dinv = jnp.where(deg > 0, lax.rsqrt(deg), 0.0)
    h = jnp.dot(x_ref[...], w1_ref[...], preferred_element_type=jnp.float32)
    p = h * dinv[:, None]
    p_ref[0] = p[:, :64]
    p_ref[1] = p[:, 64:]
    dinv_ref[...] = dinv


def _tc1(deg0, deg1, x, W1):
    return pl.pallas_call(
        _tc1_body,
        grid=(GRID,),
        in_specs=[
            pl.BlockSpec((BN,), lambda i: (i,)),
            pl.BlockSpec((BN,), lambda i: (i,)),
            pl.BlockSpec((BN, 128), lambda i: (i, 0)),
            pl.BlockSpec((128, 128), lambda i: (0, 0)),
        ],
        out_specs=[
            pl.BlockSpec((2, BN, 64), lambda i: (0, i, 0)),
            pl.BlockSpec((BN,), lambda i: (i,)),
        ],
        out_shape=[
            jax.ShapeDtypeStruct((2, N, 64), jnp.float32),
            jax.ShapeDtypeStruct((N,), jnp.float32),
        ],
    )(deg0, deg1, x, W1)


def _tc2_body(a0_ref, a1_ref, p0_ref, p1_ref, dinv_ref, b1_ref, w2_ref,
              q_ref):
    acc = jnp.concatenate(
        [a0_ref[...] + p0_ref[...], a1_ref[...] + p1_ref[...]], axis=1)
    dinv = dinv_ref[...]
    h1 = jax.nn.relu(acc * dinv[:, None] + b1_ref[...][None, :])
    q = jnp.dot(h1, w2_ref[...], preferred_element_type=jnp.float32)
    q = q * dinv[:, None]
    q_ref[0] = q[:, :32]
    q_ref[1] = q[:, 32:]


def _tc2(a0, a1, p0, p1, dinv, b1, W2):
    return pl.pallas_call(
        _tc2_body,
        grid=(GRID,),
        in_specs=[
            pl.BlockSpec((BN, 64), lambda i: (i, 0)),
            pl.BlockSpec((BN, 64), lambda i: (i, 0)),
            pl.BlockSpec((BN, 64), lambda i: (i, 0)),
            pl.BlockSpec((BN, 64), lambda i: (i, 0)),
            pl.BlockSpec((BN,), lambda i: (i,)),
            pl.BlockSpec((128,), lambda i: (0,)),
            pl.BlockSpec((128, 64), lambda i: (0, 0)),
        ],
        out_specs=pl.BlockSpec((2, BN, 32), lambda i: (0, i, 0)),
        out_shape=jax.ShapeDtypeStruct((2, N, 32), jnp.float32),
    )(a0, a1, p0, p1, dinv, b1, W2)


def _tc3_body(a0_ref, a1_ref, q0_ref, q1_ref, dinv_ref, b2_ref, out_ref):
    acc = jnp.concatenate(
        [a0_ref[...] + q0_ref[...], a1_ref[...] + q1_ref[...]], axis=1)
    o = acc * dinv_ref[...][:, None] + b2_ref[...][None, :]
    m = jnp.max(o, axis=1, keepdims=True)
    z = o - m
    out_ref[...] = z - jnp.log(jnp.sum(jnp.exp(z), axis=1, keepdims=True))


def _tc3(a0, a1, q0, q1, dinv, b2):
    return pl.pallas_call(
        _tc3_body,
        grid=(GRID,),
        in_specs=[
            pl.BlockSpec((BN, 32), lambda i: (i, 0)),
            pl.BlockSpec((BN, 32), lambda i: (i, 0)),
            pl.BlockSpec((BN, 32), lambda i: (i, 0)),
            pl.BlockSpec((BN, 32), lambda i: (i, 0)),
            pl.BlockSpec((BN,), lambda i: (i,)),
            pl.BlockSpec((64,), lambda i: (0,)),
        ],
        out_specs=pl.BlockSpec((BN, 64), lambda i: (i, 0)),
        out_shape=jax.ShapeDtypeStruct((N, 64), jnp.float32),
    )(a0, a1, q0, q1, dinv, b2)


def kernel(x, edge_index, edge_weight, W1, b1, W2, b2):
    ei = edge_index.astype(jnp.int32)
    row = ei[0]
    col = ei[1]
    ew = edge_weight.astype(jnp.float32)

    zdeg = jnp.zeros((NPAD,), jnp.float32)
    z64 = jnp.zeros((NPAD, 64), jnp.float32)
    z32 = jnp.zeros((NPAD, 32), jnp.float32)

    degp = _deg_kernel(col, ew, zdeg)
    p, dinv = _tc1(degp[0], degp[1], x, W1)
    acc1 = _agg64(row, col, ew, p[0], p[1], z64)
    q = _tc2(acc1[0, :N], acc1[1, :N], p[0], p[1], dinv, b1, W2)
    acc2 = _agg32(row, col, ew, q[0], q[1], z32)
    return _tc3(acc2[0, :N], acc2[1, :N], q[0], q[1], dinv, b2)


# speedup vs baseline: 26.6624x; 1.8854x over previous
import functools

import jax
import jax.numpy as jnp
from jax import lax
from jax.experimental import pallas as pl
from jax.experimental.pallas import tpu as pltpu
from jax.experimental.pallas import tpu_sc as plsc

N = 10000
E = 320000
NPAD = 10240
NC = 2
NS = 16
NRP = NPAD // NS
EP = E // NS

_mesh = plsc.VectorSubcoreMesh(
    core_axis_name="c", subcore_axis_name="s", num_cores=NC, num_subcores=NS
)
_sc_params = pltpu.CompilerParams(
    needs_layout_passes=False, use_tc_tiling_on_sc=False
)


CHD = 2000
EPD = E // (NC * NS)


@functools.partial(
    pl.kernel,
    out_type=jax.ShapeDtypeStruct((NC, NPAD), jnp.float32),
    mesh=_mesh,
    compiler_params=_sc_params,
    scratch_types=[
        pltpu.VMEM((CHD,), jnp.int32),
        pltpu.VMEM((CHD,), jnp.float32),
        pltpu.VMEM_SHARED((NPAD,), jnp.float32),
    ],
)
def _deg_kernel(col_h, ew_h, z_h, deg_h, col_v, ew_v, deg_sh):
    c = lax.axis_index("c")
    s = lax.axis_index("s")
    nd = NPAD // NS
    pltpu.sync_copy(z_h.at[pl.ds(s * nd, nd)], deg_sh.at[pl.ds(s * nd, nd)])
    plsc.subcore_barrier()
    base = (c * NS + s) * EPD
    for k in range(EPD // CHD):
        off = pl.multiple_of(base + k * CHD, 8)
        pltpu.sync_copy(col_h.at[pl.ds(off, CHD)], col_v)
        pltpu.sync_copy(ew_h.at[pl.ds(off, CHD)], ew_v)
        pltpu.sync_copy(ew_v, deg_sh.at[col_v], add=True)
    plsc.subcore_barrier()
    pltpu.sync_copy(deg_sh.at[pl.ds(s * nd, nd)], deg_h.at[c, pl.ds(s * nd, nd)])


def _make_agg(D, CH, IB):
    nchunks = EP // CH
    cpb = IB // CH
    nbat = EP // IB

    @functools.partial(
        pl.kernel,
        out_type=jax.ShapeDtypeStruct((NC, NPAD, D), jnp.float32),
        mesh=_mesh,
        compiler_params=_sc_params,
        scratch_types=[
            pltpu.VMEM((IB,), jnp.int32),
            pltpu.VMEM((IB,), jnp.float32),
            [pltpu.VMEM((CH,), jnp.int32)] * 2,
            [pltpu.VMEM((CH, D), jnp.float32)] * 2,
            pltpu.VMEM_SHARED((NPAD, D), jnp.float32),
            pltpu.SemaphoreType.DMA,
            [pltpu.SemaphoreType.DMA] * 2,
            [pltpu.SemaphoreType.DMA] * 2,
            [pltpu.SemaphoreType.DMA] * 2,
        ],
    )
    def agg(row_h, col_h, ew_h, p_h, z_h, acc_h, row_i, ew_i, col_b,
            rows_b, acc_sh, isem, csem, gsem, ssem):
        c = lax.axis_index("c")
        s = lax.axis_index("s")
        base = s * EP

        def colcopy(k, slot):
            return pltpu.async_copy(
                col_h.at[pl.ds(pl.multiple_of(base + k * CH, 8), CH)],
                col_b[slot], csem[slot])

        d_row = pltpu.async_copy(row_h.at[pl.ds(base, IB)], row_i, isem)
        d_ew = pltpu.async_copy(ew_h.at[pl.ds(base, IB)], ew_i, isem)
        cd = [colcopy(0, 0), colcopy(1, 1)]
        pltpu.sync_copy(z_h.at[pl.ds(s * NRP, NRP)],
                        acc_sh.at[pl.ds(s * NRP, NRP)])
        plsc.subcore_barrier()
        d_row.wait()
        d_ew.wait()

        g = [None, None]
        sc = [None, None]

        def start_gather(k, slot):
            idx = row_i.at[pl.ds((k * CH) % IB, CH)]
            return pltpu.async_copy(p_h.at[c].at[idx], rows_b[slot],
                                    gsem[slot])

        g[0] = start_gather(0, 0)
        for k in range(nchunks):
            slot = k % 2
            nslot = (k + 1) % 2
            g[slot].wait()
            if (k + 1) % cpb == 0 and k + 1 < nchunks:
                pltpu.sync_copy(row_h.at[pl.ds(base + (k + 1) * CH, IB)],
                                row_i)
            if k >= 1:
                sc[nslot].wait()
            if k + 1 < nchunks:
                g[nslot] = start_gather(k + 1, nslot)
                if k >= 1:
                    cd[nslot] = colcopy(k + 1, nslot)
            if k % cpb == 0 and k > 0:
                pltpu.sync_copy(ew_h.at[pl.ds(base + k * CH, IB)], ew_i)

            rv = rows_b[slot]
            loff = (k * CH) % IB

            @plsc.parallel_loop(0, CH, 1, unroll=4)
            def _(e):
                wv = plsc.load_gather(
                    ew_i, [jnp.full((16,), loff + e, jnp.int32)])
                for fb in range(D // 16):
                    rv[e, pl.ds(fb * 16, 16)] = rv[e, pl.ds(fb * 16, 16)] * wv

            cd[slot].wait()
            sc[slot] = pltpu.async_copy(rv, acc_sh.at[col_b[slot]],
                                        ssem[slot], add=True)
        sc[(nchunks - 1) % 2].wait()
        plsc.subcore_barrier()
        pltpu.sync_copy(acc_sh.at[pl.ds(s * NRP, NRP)],
                        acc_h.at[c, pl.ds(s * NRP, NRP)])

    return agg


_agg64 = _make_agg(64, 400, 10000)
_agg32 = _make_agg(32, 800, 20000)


BN = 512
GRID = NPAD // BN


def _tc1_body(deg0_ref, deg1_ref, x_ref, w1_ref, p_ref, dinv_ref):
    deg = deg0_ref[...] + deg1_ref[...] + 1.0
    dinv = jnp.where(deg > 0, lax.rsqrt(deg), 0.0)
    h = jnp.dot(x_ref[...], w1_ref[...], preferred_element_type=jnp.float32)
    p = h * dinv[:, None]
    p_ref[0] = p[:, :64]
    p_ref[1] = p[:, 64:]
    dinv_ref[...] = dinv


def _tc1(deg0, deg1, x, W1):
    return pl.pallas_call(
        _tc1_body,
        grid=(GRID,),
        in_specs=[
            pl.BlockSpec((BN,), lambda i: (i,)),
            pl.BlockSpec((BN,), lambda i: (i,)),
            pl.BlockSpec((BN, 128), lambda i: (i, 0)),
            pl.BlockSpec((128, 128), lambda i: (0, 0)),
        ],
        out_specs=[
            pl.BlockSpec((2, BN, 64), lambda i: (0, i, 0)),
            pl.BlockSpec((BN,), lambda i: (i,)),
        ],
        out_shape=[
            jax.ShapeDtypeStruct((2, N, 64), jnp.float32),
            jax.ShapeDtypeStruct((N,), jnp.float32),
        ],
    )(deg0, deg1, x, W1)


def _tc2_body(a0_ref, a1_ref, p0_ref, p1_ref, dinv_ref, b1_ref, w2_ref,
              q_ref):
    acc = jnp.concatenate(
        [a0_ref[...] + p0_ref[...], a1_ref[...] + p1_ref[...]], axis=1)
    dinv = dinv_ref[...]
    h1 = jax.nn.relu(acc * dinv[:, None] + b1_ref[...][None, :])
    q = jnp.dot(h1, w2_ref[...], preferred_element_type=jnp.float32)
    q = q * dinv[:, None]
    q_ref[0] = q[:, :32]
    q_ref[1] = q[:, 32:]


def _tc2(a0, a1, p0, p1, dinv, b1, W2):
    return pl.pallas_call(
        _tc2_body,
        grid=(GRID,),
        in_specs=[
            pl.BlockSpec((BN, 64), lambda i: (i, 0)),
            pl.BlockSpec((BN, 64), lambda i: (i, 0)),
            pl.BlockSpec((BN, 64), lambda i: (i, 0)),
            pl.BlockSpec((BN, 64), lambda i: (i, 0)),
            pl.BlockSpec((BN,), lambda i: (i,)),
            pl.BlockSpec((128,), lambda i: (0,)),
            pl.BlockSpec((128, 64), lambda i: (0, 0)),
        ],
        out_specs=pl.BlockSpec((2, BN, 32), lambda i: (0, i, 0)),
        out_shape=jax.ShapeDtypeStruct((2, N, 32), jnp.float32),
    )(a0, a1, p0, p1, dinv, b1, W2)


def _tc3_body(a0_ref, a1_ref, q0_ref, q1_ref, dinv_ref, b2_ref, out_ref):
    acc = jnp.concatenate(
        [a0_ref[...] + q0_ref[...], a1_ref[...] + q1_ref[...]], axis=1)
    o = acc * dinv_ref[...][:, None] + b2_ref[...][None, :]
    m = jnp.max(o, axis=1, keepdims=True)
    z = o - m
    out_ref[...] = z - jnp.log(jnp.sum(jnp.exp(z), axis=1, keepdims=True))


def _tc3(a0, a1, q0, q1, dinv, b2):
    return pl.pallas_call(
        _tc3_body,
        grid=(GRID,),
        in_specs=[
            pl.BlockSpec((BN, 32), lambda i: (i, 0)),
            pl.BlockSpec((BN, 32), lambda i: (i, 0)),
            pl.BlockSpec((BN, 32), lambda i: (i, 0)),
            pl.BlockSpec((BN, 32), lambda i: (i, 0)),
            pl.BlockSpec((BN,), lambda i: (i,)),
            pl.BlockSpec((64,), lambda i: (0,)),
        ],
        out_specs=pl.BlockSpec((BN, 64), lambda i: (i, 0)),
        out_shape=jax.ShapeDtypeStruct((N, 64), jnp.float32),
    )(a0, a1, q0, q1, dinv, b2)


def kernel(x, edge_index, edge_weight, W1, b1, W2, b2):
    ei = edge_index.astype(jnp.int32)
    row = ei[0]
    col = ei[1]
    ew = edge_weight.astype(jnp.float32)

    zdeg = jnp.zeros((NPAD,), jnp.float32)
    z64 = jnp.zeros((NPAD, 64), jnp.float32)
    z32 = jnp.zeros((NPAD, 32), jnp.float32)

    degp = _deg_kernel(col, ew, zdeg)
    p, dinv = _tc1(degp[0], degp[1], x, W1)
    acc1 = _agg64(row, col, ew, p, z64)
    q = _tc2(acc1[0, :N], acc1[1, :N], p[0], p[1], dinv, b1, W2)
    acc2 = _agg32(row, col, ew, q, z32)
    return _tc3(acc2[0, :N], acc2[1, :N], q[0], q[1], dinv, b2)


# speedup vs baseline: 27.8892x; 1.0460x over previous
import functools

import jax
import jax.numpy as jnp
from jax import lax
from jax.experimental import pallas as pl
from jax.experimental.pallas import tpu as pltpu
from jax.experimental.pallas import tpu_sc as plsc

N = 10000
E = 320000
NPAD = 10240
NC = 2
NS = 16
NRP = NPAD // NS
EP = E // NS

_mesh = plsc.VectorSubcoreMesh(
    core_axis_name="c", subcore_axis_name="s", num_cores=NC, num_subcores=NS
)
_sc_params = pltpu.CompilerParams(
    needs_layout_passes=False, use_tc_tiling_on_sc=False
)


CHD = 2000
EPD = E // (NC * NS)


@functools.partial(
    pl.kernel,
    out_type=jax.ShapeDtypeStruct((NC, NPAD), jnp.float32),
    mesh=_mesh,
    compiler_params=_sc_params,
    scratch_types=[
        pltpu.VMEM((CHD,), jnp.int32),
        pltpu.VMEM((CHD,), jnp.float32),
        pltpu.VMEM_SHARED((NPAD,), jnp.float32),
    ],
)
def _deg_kernel(col_h, ew_h, z_h, deg_h, col_v, ew_v, deg_sh):
    c = lax.axis_index("c")
    s = lax.axis_index("s")
    nd = NPAD // NS
    pltpu.sync_copy(z_h.at[pl.ds(s * nd, nd)], deg_sh.at[pl.ds(s * nd, nd)])
    plsc.subcore_barrier()
    base = (c * NS + s) * EPD
    for k in range(EPD // CHD):
        off = pl.multiple_of(base + k * CHD, 8)
        pltpu.sync_copy(col_h.at[pl.ds(off, CHD)], col_v)
        pltpu.sync_copy(ew_h.at[pl.ds(off, CHD)], ew_v)
        pltpu.sync_copy(ew_v, deg_sh.at[col_v], add=True)
    plsc.subcore_barrier()
    pltpu.sync_copy(deg_sh.at[pl.ds(s * nd, nd)], deg_h.at[c, pl.ds(s * nd, nd)])


def _make_agg(D, CH, IB):
    nchunks = EP // CH
    cpb = IB // CH
    nbat = EP // IB

    @functools.partial(
        pl.kernel,
        out_type=jax.ShapeDtypeStruct((NC, NPAD, D), jnp.float32),
        mesh=_mesh,
        compiler_params=_sc_params,
        scratch_types=[
            pltpu.VMEM((IB,), jnp.int32),
            pltpu.VMEM((IB,), jnp.float32),
            [pltpu.VMEM((CH,), jnp.int32)] * 2,
            [pltpu.VMEM((CH, D), jnp.float32)] * 2,
            pltpu.VMEM_SHARED((NPAD, D), jnp.float32),
            pltpu.SemaphoreType.DMA,
            [pltpu.SemaphoreType.DMA] * 2,
            [pltpu.SemaphoreType.DMA] * 2,
            [pltpu.SemaphoreType.DMA] * 2,
        ],
    )
    def agg(row_h, col_h, ew_h, p_h, z_h, acc_h, row_i, ew_i, col_b,
            rows_b, acc_sh, isem, csem, gsem, ssem):
        c = lax.axis_index("c")
        s = lax.axis_index("s")
        base = s * EP

        def colcopy(k, slot):
            return pltpu.async_copy(
                col_h.at[pl.ds(pl.multiple_of(base + k * CH, 8), CH)],
                col_b[slot], csem[slot])

        d_row = pltpu.async_copy(row_h.at[pl.ds(base, IB)], row_i, isem)
        d_ew = pltpu.async_copy(ew_h.at[pl.ds(base, IB)], ew_i, isem)
        cd = [colcopy(0, 0), colcopy(1, 1)]
        pltpu.sync_copy(z_h.at[pl.ds(s * NRP, NRP)],
                        acc_sh.at[pl.ds(s * NRP, NRP)])
        plsc.subcore_barrier()
        d_row.wait()
        d_ew.wait()

        g = [None, None]
        sc = [None, None]

        def start_gather(k, slot):
            idx = row_i.at[pl.ds((k * CH) % IB, CH)]
            return pltpu.async_copy(p_h.at[c].at[idx], rows_b[slot],
                                    gsem[slot])

        g[0] = start_gather(0, 0)
        for k in range(nchunks):
            slot = k % 2
            nslot = (k + 1) % 2
            g[slot].wait()
            if (k + 1) % cpb == 0 and k + 1 < nchunks:
                pltpu.sync_copy(row_h.at[pl.ds(base + (k + 1) * CH, IB)],
                                row_i)
            if k >= 1:
                sc[nslot].wait()
            if k + 1 < nchunks:
                g[nslot] = start_gather(k + 1, nslot)
                if k >= 1:
                    cd[nslot] = colcopy(k + 1, nslot)
            if k % cpb == 0 and k > 0:
                pltpu.sync_copy(ew_h.at[pl.ds(base + k * CH, IB)], ew_i)

            rv = rows_b[slot]
            loff = (k * CH) % IB

            @plsc.parallel_loop(0, CH, 1, unroll=8)
            def _(e):
                wv = plsc.load_gather(
                    ew_i, [jnp.full((16,), loff + e, jnp.int32)])
                for fb in range(D // 16):
                    rv[e, pl.ds(fb * 16, 16)] = rv[e, pl.ds(fb * 16, 16)] * wv

            cd[slot].wait()
            sc[slot] = pltpu.async_copy(rv, acc_sh.at[col_b[slot]],
                                        ssem[slot], add=True)
        sc[(nchunks - 1) % 2].wait()
        plsc.subcore_barrier()
        pltpu.sync_copy(acc_sh.at[pl.ds(s * NRP, NRP)],
                        acc_h.at[c, pl.ds(s * NRP, NRP)])

    return agg


_agg64 = _make_agg(64, 400, 10000)
_agg32 = _make_agg(32, 800, 20000)


BN = 512
GRID = NPAD // BN


def _tc1_body(deg0_ref, deg1_ref, x_ref, w1_ref, p_ref, dinv_ref):
    deg = deg0_ref[...] + deg1_ref[...] + 1.0
    dinv = jnp.where(deg > 0, lax.rsqrt(deg), 0.0)
    h = jnp.dot(x_ref[...], w1_ref[...], preferred_element_type=jnp.float32)
    p = h * dinv[:, None]
    p_ref[0] = p[:, :64]
    p_ref[1] = p[:, 64:]
    dinv_ref[...] = dinv


def _tc1(deg0, deg1, x, W1):
    return pl.pallas_call(
        _tc1_body,
        grid=(GRID,),
        in_specs=[
            pl.BlockSpec((BN,), lambda i: (i,)),
            pl.BlockSpec((BN,), lambda i: (i,)),
            pl.BlockSpec((BN, 128), lambda i: (i, 0)),
            pl.BlockSpec((128, 128), lambda i: (0, 0)),
        ],
        out_specs=[
            pl.BlockSpec((2, BN, 64), lambda i: (0, i, 0)),
            pl.BlockSpec((BN,), lambda i: (i,)),
        ],
        out_shape=[
            jax.ShapeDtypeStruct((2, N, 64), jnp.float32),
            jax.ShapeDtypeStruct((N,), jnp.float32),
        ],
    )(deg0, deg1, x, W1)


def _tc2_body(a_ref, p_ref, dinv_ref, b1_ref, w2_ref, q_ref):
    acc = jnp.concatenate(
        [a_ref[0] + p_ref[0], a_ref[1] + p_ref[1]], axis=1)
    dinv = dinv_ref[...]
    h1 = jax.nn.relu(acc * dinv[:, None] + b1_ref[...][None, :])
    q = jnp.dot(h1, w2_ref[...], preferred_element_type=jnp.float32)
    q = q * dinv[:, None]
    q_ref[0] = q[:, :32]
    q_ref[1] = q[:, 32:]


def _tc2(a, p, dinv, b1, W2):
    return pl.pallas_call(
        _tc2_body,
        grid=(GRID,),
        in_specs=[
            pl.BlockSpec((2, BN, 64), lambda i: (0, i, 0)),
            pl.BlockSpec((2, BN, 64), lambda i: (0, i, 0)),
            pl.BlockSpec((BN,), lambda i: (i,)),
            pl.BlockSpec((128,), lambda i: (0,)),
            pl.BlockSpec((128, 64), lambda i: (0, 0)),
        ],
        out_specs=pl.BlockSpec((2, BN, 32), lambda i: (0, i, 0)),
        out_shape=jax.ShapeDtypeStruct((2, N, 32), jnp.float32),
    )(a, p, dinv, b1, W2)


def _tc3_body(a_ref, q_ref, dinv_ref, b2_ref, out_ref):
    acc = jnp.concatenate(
        [a_ref[0] + q_ref[0], a_ref[1] + q_ref[1]], axis=1)
    o = acc * dinv_ref[...][:, None] + b2_ref[...][None, :]
    m = jnp.max(o, axis=1, keepdims=True)
    z = o - m
    out_ref[...] = z - jnp.log(jnp.sum(jnp.exp(z), axis=1, keepdims=True))


def _tc3(a, q, dinv, b2):
    return pl.pallas_call(
        _tc3_body,
        grid=(GRID,),
        in_specs=[
            pl.BlockSpec((2, BN, 32), lambda i: (0, i, 0)),
            pl.BlockSpec((2, BN, 32), lambda i: (0, i, 0)),
            pl.BlockSpec((BN,), lambda i: (i,)),
            pl.BlockSpec((64,), lambda i: (0,)),
        ],
        out_specs=pl.BlockSpec((BN, 64), lambda i: (i, 0)),
        out_shape=jax.ShapeDtypeStruct((N, 64), jnp.float32),
    )(a, q, dinv, b2)


def kernel(x, edge_index, edge_weight, W1, b1, W2, b2):
    ei = edge_index.astype(jnp.int32)
    row = ei[0]
    col = ei[1]
    ew = edge_weight.astype(jnp.float32)

    zdeg = jnp.zeros((NPAD,), jnp.float32)
    z64 = jnp.zeros((NPAD, 64), jnp.float32)
    z32 = jnp.zeros((NPAD, 32), jnp.float32)

    degp = _deg_kernel(col, ew, zdeg)
    p, dinv = _tc1(degp[0], degp[1], x, W1)
    acc1 = _agg64(row, col, ew, p, z64)
    q = _tc2(acc1, p, dinv, b1, W2)
    acc2 = _agg32(row, col, ew, q, z32)
    return _tc3(acc2, q, dinv, b2)
